# baseline (speedup 1.0000x reference)
import jax
import jax.numpy as jnp
from jax.experimental import pallas as pl
from jax.experimental.pallas import tpu as pltpu

N_GRAPHS = 1024
BLK = 1024
WIN = 128
N_CHUNKS = (N_GRAPHS + 8 + WIN - 1) // WIN


def _fused_kernel(base_ref, smax_ref, seg_ref, h_ref, w1_ref, b1_ref, w2_ref,
                  b2_ref, wp_ref, bp_ref, out_ref):
    i = pl.program_id(0)

    @pl.when(i == 0)
    def _():
        out_ref[...] = jnp.zeros_like(out_ref)

    h = h_ref[...]
    hid = jnp.maximum(
        jnp.dot(h, w1_ref[...], preferred_element_type=jnp.float32)
        + b1_ref[...], 0.0)
    logit = jnp.dot(hid, w2_ref[...],
                    preferred_element_type=jnp.float32) + b2_ref[0, 0]
    gate = jax.nn.sigmoid(logit)
    feat = jnp.dot(h, wp_ref[...],
                   preferred_element_type=jnp.float32) + bp_ref[...]
    gated = (gate * feat).astype(jnp.bfloat16)

    seg = seg_ref[0, 0, :]
    base = (base_ref[i] // 8) * 8
    smax = smax_ref[i]
    iota = jax.lax.broadcasted_iota(jnp.int32, (WIN, BLK), 0)

    def chunk(c):
        start = base + c * WIN
        onehot = (iota == (seg - start)[None, :]).astype(jnp.bfloat16)
        out_ref[pl.ds(start, WIN), :] += jnp.dot(
            onehot, gated, preferred_element_type=jnp.float32)

    chunk(0)
    for c in range(1, N_CHUNKS):
        @pl.when(smax >= base + c * WIN)
        def _(c=c):
            chunk(c)


@jax.jit
def kernel(h_nodes, batch, W1, b1, W2, b2, Wp, bp):
    n, d = h_nodes.shape
    out_dim = Wp.shape[0]
    hidden = W1.shape[0]
    nblk = -(-n // BLK)
    pad = nblk * BLK - n

    h_p = jnp.pad(h_nodes, ((0, pad), (0, 0))).astype(jnp.bfloat16)
    seg_flat = jnp.pad(batch.astype(jnp.int32), (0, pad),
                       constant_values=N_GRAPHS)
    seg = seg_flat.reshape(nblk, 1, BLK)
    bases = seg_flat[::BLK]
    smaxs = seg_flat[BLK - 1::BLK]

    w1t = W1.T.astype(jnp.bfloat16)
    b1r = b1.reshape(1, hidden)
    w2t = W2.T
    b2r = b2.reshape(1, 1)
    wpt = Wp.T.astype(jnp.bfloat16)
    bpr = bp.reshape(1, out_dim)

    out = pl.pallas_call(
        _fused_kernel,
        grid_spec=pltpu.PrefetchScalarGridSpec(
            num_scalar_prefetch=2,
            grid=(nblk,),
            in_specs=[
                pl.BlockSpec((1, 1, BLK), lambda i, b, s: (i, 0, 0)),
                pl.BlockSpec((BLK, d), lambda i, b, s: (i, 0)),
                pl.BlockSpec((d, hidden), lambda i, b, s: (0, 0)),
                pl.BlockSpec((1, hidden), lambda i, b, s: (0, 0)),
                pl.BlockSpec((hidden, 1), lambda i, b, s: (0, 0)),
                pl.BlockSpec((1, 1), lambda i, b, s: (0, 0)),
                pl.BlockSpec((d, out_dim), lambda i, b, s: (0, 0)),
                pl.BlockSpec((1, out_dim), lambda i, b, s: (0, 0)),
            ],
            out_specs=pl.BlockSpec((N_GRAPHS + WIN, out_dim),
                                   lambda i, b, s: (0, 0)),
        ),
        out_shape=jax.ShapeDtypeStruct((N_GRAPHS + WIN, out_dim), jnp.float32),
    )(bases, smaxs, seg, h_p, w1t, b1r, w2t, b2r, wpt, bpr)
    return out[:N_GRAPHS]


# speedup vs baseline: 1.2472x; 1.2472x over previous
import jax
import jax.numpy as jnp
from jax.experimental import pallas as pl
from jax.experimental.pallas import tpu as pltpu

N_GRAPHS = 1024
BLK = 2048
WIN = 128
N_CHUNKS = (N_GRAPHS + 8 + WIN - 1) // WIN


def _fused_kernel(base_ref, smax_ref, seg_ref, h_ref, wcat_ref, b1_ref,
                  w2_ref, b2_ref, bp_ref, out_ref):
    i = pl.program_id(0)

    @pl.when(i == 0)
    def _():
        out_ref[...] = jnp.zeros_like(out_ref)

    h = h_ref[...]
    x = jnp.dot(h, wcat_ref[...],
                preferred_element_type=jnp.float32)
    feat = x[:, :128] + bp_ref[...]
    hid = jnp.maximum(x[:, 128:192] + b1_ref[...], 0.0)
    logit = jnp.dot(hid.astype(jnp.bfloat16), w2_ref[...],
                    preferred_element_type=jnp.float32) + b2_ref[0, 0]
    gate = jax.nn.sigmoid(logit)
    gated = (gate * feat).astype(jnp.bfloat16)

    seg = seg_ref[0, 0, :]
    base = (base_ref[i] // 8) * 8
    smax = smax_ref[i]
    iota = jax.lax.broadcasted_iota(jnp.int32, (WIN, BLK), 0)

    def chunk(c):
        start = base + c * WIN
        onehot = (iota == (seg - start)[None, :]).astype(jnp.bfloat16)
        out_ref[pl.ds(start, WIN), :] += jnp.dot(
            onehot, gated, preferred_element_type=jnp.float32)

    chunk(0)
    for c in range(1, N_CHUNKS):
        @pl.when(smax >= base + c * WIN)
        def _(c=c):
            chunk(c)


@jax.jit
def kernel(h_nodes, batch, W1, b1, W2, b2, Wp, bp):
    n, d = h_nodes.shape
    out_dim = Wp.shape[0]
    hidden = W1.shape[0]
    nblk = -(-n // BLK)
    pad = nblk * BLK - n

    h_p = jnp.pad(h_nodes, ((0, pad), (0, 0))).astype(jnp.bfloat16)
    seg_flat = jnp.pad(batch.astype(jnp.int32), (0, pad),
                       constant_values=N_GRAPHS)
    seg = seg_flat.reshape(nblk, 1, BLK)
    bases = seg_flat[::BLK]
    smaxs = seg_flat[BLK - 1::BLK]

    wcat = jnp.zeros((d, 256), jnp.float32)
    wcat = wcat.at[:, :out_dim].set(Wp.T).at[:, 128:128 + hidden].set(W1.T)
    wcat = wcat.astype(jnp.bfloat16)
    b1r = b1.reshape(1, hidden)
    w2t = W2.T.astype(jnp.bfloat16)
    b2r = b2.reshape(1, 1)
    bpr = bp.reshape(1, out_dim)

    out = pl.pallas_call(
        _fused_kernel,
        grid_spec=pltpu.PrefetchScalarGridSpec(
            num_scalar_prefetch=2,
            grid=(nblk,),
            in_specs=[
                pl.BlockSpec((1, 1, BLK), lambda i, b, s: (i, 0, 0)),
                pl.BlockSpec((BLK, d), lambda i, b, s: (i, 0)),
                pl.BlockSpec((d, 256), lambda i, b, s: (0, 0)),
                pl.BlockSpec((1, hidden), lambda i, b, s: (0, 0)),
                pl.BlockSpec((hidden, 1), lambda i, b, s: (0, 0)),
                pl.BlockSpec((1, 1), lambda i, b, s: (0, 0)),
                pl.BlockSpec((1, out_dim), lambda i, b, s: (0, 0)),
            ],
            out_specs=pl.BlockSpec((N_GRAPHS + WIN, out_dim),
                                   lambda i, b, s: (0, 0)),
        ),
        out_shape=jax.ShapeDtypeStruct((N_GRAPHS + WIN, out_dim), jnp.float32),
    )(bases, smaxs, seg, h_p, wcat, b1r, w2t, b2r, bpr)
    return out[:N_GRAPHS]
